# baseline (speedup 1.0000x reference)
import jax
import jax.numpy as jnp
from jax import lax
from jax.experimental import pallas as pl
from jax.experimental.pallas import tpu as pltpu
from jax.experimental.pallas import tpu_sc as plsc

_H = 32
_T = 2209
_N = 576 * 576
_NC, _NS, _L = 2, 16, 16
_NW = _NC * _NS
_NPW = _N // _NW
_CHUNKS = _NPW // _L
_SUB = 384
_CPS = _SUB // _L
_NBANDS = _NPW // _SUB


def _body(tab_hbm, idx_hbm, out_hbm, tab_v, idx_v, ob0, ob1, sem0, sem1):
    w = lax.axis_index("s") * _NC + lax.axis_index("c")
    base = w * _NPW
    pltpu.sync_copy(tab_hbm, tab_v)
    pltpu.sync_copy(idx_hbm.at[pl.ds(base, _NPW)], idx_v)

    def _fill(r, ob):
        @plsc.parallel_loop(0, _CPS, 1, unroll=4)
        def _chunk(c):
            iv = idx_v[pl.ds(r * _SUB + c * _L, _L)]
            s = pl.ds(c * _L, _L)
            for h in range(_H):
                ob[h, s] = plsc.load_gather(tab_v, [iv + (h * _T)])

    def _dst(r):
        return out_hbm.at[:, pl.ds(base + r * _SUB, _SUB)]

    def _outer(r2, carry):
        @pl.when(r2 != 0)
        def _():
            pltpu.make_async_copy(ob0, _dst(0), sem0).wait()

        _fill(2 * r2, ob0)
        pltpu.async_copy(ob0, _dst(2 * r2), sem0)

        @pl.when(r2 != 0)
        def _():
            pltpu.make_async_copy(ob1, _dst(0), sem1).wait()

        _fill(2 * r2 + 1, ob1)
        pltpu.async_copy(ob1, _dst(2 * r2 + 1), sem1)
        return carry

    lax.fori_loop(0, _NBANDS // 2, _outer, 0)
    pltpu.make_async_copy(ob0, _dst(0), sem0).wait()
    _fill(_NBANDS - 1, ob0)
    pltpu.async_copy(ob0, _dst(_NBANDS - 1), sem0)
    pltpu.make_async_copy(ob0, _dst(0), sem0).wait()
    pltpu.make_async_copy(ob1, _dst(0), sem1).wait()


def kernel(relative_position_bias_table, relative_index):
    tab_t = relative_position_bias_table.T.reshape(-1)
    idx_flat = relative_index.reshape(-1)
    mesh = plsc.VectorSubcoreMesh(core_axis_name="c", subcore_axis_name="s")
    out = pl.kernel(
        _body,
        out_type=jax.ShapeDtypeStruct((_H, _N), jnp.float32),
        mesh=mesh,
        scratch_types=[
            pltpu.VMEM((_H * _T,), jnp.float32),
            pltpu.VMEM((_NPW,), jnp.int32),
            pltpu.VMEM((_H, _SUB), jnp.float32),
            pltpu.VMEM((_H, _SUB), jnp.float32),
            pltpu.SemaphoreType.DMA,
            pltpu.SemaphoreType.DMA,
        ],
        compiler_params=pltpu.CompilerParams(needs_layout_passes=False),
    )(tab_t, idx_flat)
    return out.reshape(_H, 576, 576)


# speedup vs baseline: 2.3744x; 2.3744x over previous
import jax
import jax.numpy as jnp
from jax import lax
from jax.experimental import pallas as pl
from jax.experimental.pallas import tpu as pltpu
from jax.experimental.pallas import tpu_sc as plsc

_H = 32
_T = 2209
_R = 576
_C = 576
_NC, _NS, _L = 2, 16, 16
_HG = 8
_WPG = 8
_SR = 8
_SE = _SR * _C
_NST = _R // _SR
_SPW = _NST // _WPG
_CPR = _C // _L


def _body(tab_hbm, idx_hbm, out_hbm,
          tab_v, ix0, ix1, ob0, ob1, semi0, semi1, semo0, semo1):
    w = lax.axis_index("s") * _NC + lax.axis_index("c")
    hg = w // _WPG
    wk = w % _WPG
    hbase = pl.multiple_of(hg * _HG, _HG)
    pltpu.sync_copy(tab_hbm.at[pl.ds(hbase, _HG), :], tab_v)

    def _idx_src(j):
        off = pl.multiple_of((wk + j * _WPG) * _SE, _SE)
        return idx_hbm.at[pl.ds(off, _SE)]

    def _dst(j):
        r0 = pl.multiple_of((wk + j * _WPG) * _SR, _SR)
        return out_hbm.at[pl.ds(hbase, _HG), pl.ds(r0, _SR), :]

    def _fill(ix, ob):
        for rr in range(_SR):
            @plsc.parallel_loop(0, _CPR, 1, unroll=2)
            def _chunk(c):
                iv = ix[pl.ds(rr * _C + c * _L, _L)]
                s = pl.ds(c * _L, _L)
                for hl in range(_HG):
                    hv = jnp.full((_L,), hl, jnp.int32)
                    ob[hl, rr, s] = plsc.load_gather(tab_v, [hv, iv])

    pltpu.async_copy(_idx_src(0), ix0, semi0)

    def _stripe(j, ix, semi, ob, semo, nxt_ix, nxt_semi, first):
        pltpu.make_async_copy(_idx_src(j), ix, semi).wait()
        pltpu.async_copy(_idx_src(j + 1), nxt_ix, nxt_semi)

        @pl.when(jnp.logical_not(first))
        def _():
            pltpu.make_async_copy(ob, _dst(j), semo).wait()

        _fill(ix, ob)
        pltpu.async_copy(ob, _dst(j), semo)

    def _pair(jp, carry):
        _stripe(2 * jp, ix0, semi0, ob0, semo0, ix1, semi1, jp == 0)
        _stripe(2 * jp + 1, ix1, semi1, ob1, semo1, ix0, semi0, jp == 0)
        return carry

    lax.fori_loop(0, (_SPW - 1) // 2, _pair, 0)

    jt = _SPW - 1
    pltpu.make_async_copy(_idx_src(jt), ix0, semi0).wait()
    pltpu.make_async_copy(ob0, _dst(jt), semo0).wait()
    _fill(ix0, ob0)
    pltpu.async_copy(ob0, _dst(jt), semo0)
    pltpu.make_async_copy(ob0, _dst(jt), semo0).wait()
    pltpu.make_async_copy(ob1, _dst(jt), semo1).wait()


def kernel(relative_position_bias_table, relative_index):
    tab_t = relative_position_bias_table.T
    idx_flat = relative_index.reshape(-1)
    mesh = plsc.VectorSubcoreMesh(core_axis_name="c", subcore_axis_name="s")
    return pl.kernel(
        _body,
        out_type=jax.ShapeDtypeStruct((_H, _R, _C), jnp.float32),
        mesh=mesh,
        scratch_types=[
            pltpu.VMEM((_HG, _T), jnp.float32),
            pltpu.VMEM((_SE,), jnp.int32),
            pltpu.VMEM((_SE,), jnp.int32),
            pltpu.VMEM((_HG, _SR, _C), jnp.float32),
            pltpu.VMEM((_HG, _SR, _C), jnp.float32),
            pltpu.SemaphoreType.DMA,
            pltpu.SemaphoreType.DMA,
            pltpu.SemaphoreType.DMA,
            pltpu.SemaphoreType.DMA,
        ],
        compiler_params=pltpu.CompilerParams(needs_layout_passes=False),
    )(tab_t, idx_flat)
